# baseline (speedup 1.0000x reference)
import functools

import jax
import jax.numpy as jnp
from jax import lax
from jax.experimental import pallas as pl
from jax.experimental.pallas import tpu as pltpu
from jax.experimental.pallas import tpu_sc as plsc

VOCAB = 1000000
D = 32
B = 16384
F = 26
N = B * F

NC = 2
NS = 16
NW = NC * NS
ROWS_PER_W = N // NW

V4 = VOCAB // 4
CH = 256
N_CH = ROWS_PER_W // CH
OUT_WORDS_PER_CH = CH * D

_mesh = plsc.VectorSubcoreMesh(
    core_axis_name="c", subcore_axis_name="s", num_cores=NC, num_subcores=NS
)


@functools.partial(
    pl.kernel,
    out_type=jax.ShapeDtypeStruct((N * D,), jnp.float32),
    mesh=_mesh,
    scratch_types=[
        pltpu.VMEM((ROWS_PER_W,), jnp.int32),
        pltpu.VMEM((ROWS_PER_W,), jnp.int32),
        pltpu.VMEM((CH, 128), jnp.float32),
        pltpu.VMEM((CH, 128), jnp.float32),
        pltpu.VMEM((OUT_WORDS_PER_CH,), jnp.float32),
        pltpu.VMEM((OUT_WORDS_PER_CH,), jnp.float32),
        pltpu.SemaphoreType.DMA,
        pltpu.SemaphoreType.DMA,
        pltpu.SemaphoreType.DMA,
        pltpu.SemaphoreType.DMA,
    ],
    compiler_params=pltpu.CompilerParams(
        use_tc_tiling_on_sc=True, needs_layout_passes=False
    ),
)
def _embed_kernel(
    x_hbm, w_hbm, out_hbm, jbuf, sbuf, raw0, raw1, outb0, outb1, g0, g1, s0, s1
):
    wid = lax.axis_index("s") * NC + lax.axis_index("c")
    base = wid * ROWS_PER_W
    pltpu.sync_copy(x_hbm.at[pl.ds(base, ROWS_PER_W)], jbuf)

    @pl.loop(0, ROWS_PER_W // 16, unroll=8)
    def _prep(t):
        iv = jbuf[pl.ds(t * 16, 16)]
        jbuf[pl.ds(t * 16, 16)] = lax.shift_right_logical(iv, 2)
        sbuf[pl.ds(t * 16, 16)] = lax.shift_left(lax.bitwise_and(iv, 3), 5)

    raws = (raw0, raw1)
    outbs = (outb0, outb1)
    gsems = (g0, g1)
    ssems = (s0, s1)
    lanes = lax.iota(jnp.int32, 16)

    def fire_gather(c, slot):
        return pltpu.async_copy(
            w_hbm.at[jbuf.at[pl.ds(c * CH, CH)]], raws[slot], gsems[slot]
        )

    def fire_store(c, slot):
        return pltpu.async_copy(
            outbs[slot],
            out_hbm.at[pl.ds((base + c * CH) * D, OUT_WORDS_PER_CH)],
            ssems[slot],
        )

    def wait_store(c, slot):
        pltpu.make_async_copy(
            outbs[slot],
            out_hbm.at[pl.ds((base + c * CH) * D, OUT_WORDS_PER_CH)],
            ssems[slot],
        ).wait()

    def wait_gather(c, slot):
        pltpu.make_async_copy(
            w_hbm.at[jbuf.at[pl.ds(c * CH, CH)]], raws[slot], gsems[slot]
        ).wait()

    def extract(c, slot):
        raw = raws[slot]
        outb = outbs[slot]

        @pl.loop(0, CH // 16)
        def _rows(t):
            rowv = t * 16 + lanes
            scolv = sbuf[pl.ds(c * CH + t * 16, 16)]
            dstv = lax.shift_left(rowv, 5)
            for w in range(D):
                gv = plsc.load_gather(raw, [rowv, scolv + w])
                plsc.store_scatter(outb, [dstv + w], gv)

    gather0 = fire_gather(0, 0)

    @pl.loop(0, N_CH // 2)
    def _pipe(p):
        gc0 = 2 * p
        gc1 = 2 * p + 1
        wait_gather(gc0, 0)

        @pl.when(p > 0)
        def _():
            wait_store(gc0 - 2, 0)

        fire_gather(gc1, 1)
        extract(gc0, 0)
        fire_store(gc0, 0)
        wait_gather(gc1, 1)

        @pl.when(p > 0)
        def _():
            wait_store(gc1 - 2, 1)

        @pl.when(p + 1 < N_CH // 2)
        def _():
            fire_gather(gc0 + 2, 0)

        extract(gc1, 1)
        fire_store(gc1, 1)

    wait_store(N_CH - 2, 0)
    wait_store(N_CH - 1, 1)


def kernel(x, weight):
    w128 = weight.reshape(V4, 128)
    x_flat = x.reshape(-1).astype(jnp.int32)
    out = _embed_kernel(x_flat, w128)
    return out.reshape(B, F, D)


# speedup vs baseline: 1.3816x; 1.3816x over previous
import functools

import jax
import jax.numpy as jnp
from jax import lax
from jax.experimental import pallas as pl
from jax.experimental.pallas import tpu as pltpu
from jax.experimental.pallas import tpu_sc as plsc

VOCAB = 1000000
D = 32
B = 16384
F = 26

NC = 2
NS = 16
NW = NC * NS
BW = B // NW

_mesh = plsc.VectorSubcoreMesh(
    core_axis_name="c", subcore_axis_name="s", num_cores=NC, num_subcores=NS
)


@functools.partial(
    pl.kernel,
    out_type=jax.ShapeDtypeStruct((F, D, B), jnp.float32),
    mesh=_mesh,
    scratch_types=[
        pltpu.VMEM((BW,), jnp.int32),
        pltpu.VMEM((BW,), jnp.int32),
        pltpu.VMEM((BW, D), jnp.float32),
        pltpu.VMEM((BW, D), jnp.float32),
        pltpu.VMEM((D, BW), jnp.float32),
        pltpu.VMEM((D, BW), jnp.float32),
        pltpu.SemaphoreType.DMA,
        pltpu.SemaphoreType.DMA,
        pltpu.SemaphoreType.DMA,
        pltpu.SemaphoreType.DMA,
    ],
    compiler_params=pltpu.CompilerParams(
        use_tc_tiling_on_sc=False, needs_layout_passes=False
    ),
)
def _embed_kernel(
    xt_hbm, w_hbm, out_hbm, idx0, idx1, raw0, raw1, tb0, tb1, g0, g1, s0, s1
):
    wid = lax.axis_index("s") * NC + lax.axis_index("c")
    boff = wid * BW

    idxs = (idx0, idx1)
    raws = (raw0, raw1)
    tbs = (tb0, tb1)
    gsems = (g0, g1)
    ssems = (s0, s1)
    lanes = lax.iota(jnp.int32, 16)

    def stage_idx(f, slot):
        pltpu.sync_copy(xt_hbm.at[f, pl.ds(boff, BW)], idxs[slot])

    def fire_gather(slot):
        return pltpu.async_copy(w_hbm.at[idxs[slot]], raws[slot], gsems[slot])

    def wait_gather(slot):
        pltpu.make_async_copy(
            w_hbm.at[idxs[slot]], raws[slot], gsems[slot]
        ).wait()

    def fire_store(f, slot):
        return pltpu.async_copy(
            tbs[slot], out_hbm.at[f, :, pl.ds(boff, BW)], ssems[slot]
        )

    def wait_store(f, slot):
        pltpu.make_async_copy(
            tbs[slot], out_hbm.at[f, :, pl.ds(boff, BW)], ssems[slot]
        ).wait()

    def transpose(slot):
        raw = raws[slot]
        tb = tbs[slot]

        @pl.loop(0, BW // 16)
        def _grp(t):
            rowv = t * 16 + lanes
            for d in range(D):
                tb[d, pl.ds(t * 16, 16)] = plsc.load_gather(
                    raw, [rowv, jnp.full((16,), d, jnp.int32)]
                )

    stage_idx(0, 0)
    fire_gather(0)

    @pl.loop(0, F // 2)
    def _pipe(p):
        f0 = 2 * p
        f1 = 2 * p + 1

        wait_gather(0)
        stage_idx(f1, 1)

        @pl.when(p > 0)
        def _():
            wait_store(f1 - 2, 1)

        fire_gather(1)

        @pl.when(p > 0)
        def _():
            wait_store(f0 - 2, 0)

        transpose(0)
        fire_store(f0, 0)

        wait_gather(1)

        @pl.when(p + 1 < F // 2)
        def _():
            stage_idx(f0 + 2, 0)
            fire_gather(0)

        transpose(1)
        fire_store(f1, 1)

    wait_store(F - 2, 0)
    wait_store(F - 1, 1)


def kernel(x, weight):
    xt = jnp.transpose(x).astype(jnp.int32)
    out3 = _embed_kernel(xt, weight)
    return jnp.transpose(out3, (2, 0, 1))
